# Initial kernel scaffold; baseline (speedup 1.0000x reference)
import jax
import jax.numpy as jnp
from jax.experimental import pallas as pl


def kernel(X, edge_idx, batch_id, chain_encoding, virtual_frame_num):
    raise NotImplementedError("write your pallas kernel here")


# speedup vs baseline: 1.9442x; 1.9442x over previous
import functools

import jax
import jax.numpy as jnp
import numpy as np
from jax import lax
from jax.experimental import pallas as pl
from jax.experimental.pallas import tpu as pltpu
from jax.experimental.pallas import tpu_sc as plsc

_N = 50000
_E = 800000
_NUM_RBF = 16
_NB = 2000
_EB = 2000
_DTAB = 32

_MU_STEP = np.float32(20.0 / (_NUM_RBF - 1))
_INV_SIG = np.float32(_NUM_RBF / 20.0)
_FREQ_COEF = np.float32(-2.0 * np.log(10000.0) / _NUM_RBF)


def _mu_row():
    i = lax.broadcasted_iota(jnp.int32, (1, _NUM_RBF), 1).astype(jnp.float32)
    return i * _MU_STEP


def _freq_row():
    i = lax.broadcasted_iota(jnp.int32, (1, 8), 1).astype(jnp.float32)
    return jnp.exp(i * _FREQ_COEF)


def _decouple(proj):
    nsq = jnp.sum(proj * proj, axis=-1, keepdims=True)
    norm = jnp.sqrt(nsq)
    direct = proj / (norm + 1e-6)
    z = (norm - _mu_row()) * _INV_SIG
    rbf = jnp.exp(-(z * z))
    return direct, rbf


def _node_body(x_ref, xprev_ref, r_ref, t_ref, v_ref, f_ref):
    x = x_ref[...]
    xp = xprev_ref[...]
    ca = x[:, 3:6]
    n = x[:, 0:3] - ca
    c = x[:, 6:9] - ca
    cx, cy, cz = c[:, 0:1], c[:, 1:2], c[:, 2:3]
    eps = 1e-20
    norm1 = jnp.sqrt(eps + cx * cx + cy * cy)
    sin1 = -cy / norm1
    cos1 = cx / norm1
    norm2 = jnp.sqrt(eps + cx * cx + cy * cy + cz * cz)
    sin2 = cz / norm2
    cos2 = jnp.sqrt(cx * cx + cy * cy) / norm2
    c1 = [[cos1, -sin1, 0.0], [sin1, cos1, 0.0], [0.0, 0.0, 1.0]]
    c2 = [[cos2, 0.0, sin2], [0.0, 1.0, 0.0], [-sin2, 0.0, cos2]]
    cr = [[sum(c2[i][j] * c1[j][k] for j in range(3)) for k in range(3)]
          for i in range(3)]
    nvec = [n[:, 0:1], n[:, 1:2], n[:, 2:3]]
    nrot = [sum(cr[i][j] * nvec[j] for j in range(3)) for i in range(3)]
    norm3 = jnp.sqrt(eps + nrot[1] * nrot[1] + nrot[2] * nrot[2])
    sinn = -nrot[2] / norm3
    cosn = nrot[1] / norm3
    nm = [[1.0, 0.0, 0.0], [0.0, cosn, -sinn], [0.0, sinn, cosn]]
    rots = [[sum(nm[i][j] * cr[j][k] for j in range(3)) for k in range(3)]
            for i in range(3)]
    ones = jnp.ones_like(cx)
    rcols = [rots[b][a] * ones for a in range(3) for b in range(3)]
    r9 = jnp.concatenate(rcols, axis=-1)
    r_ref[...] = r9
    t_ref[...] = ca
    rrow = [jnp.concatenate([rots[0][j] * ones, rots[1][j] * ones,
                             rots[2][j] * ones], axis=-1) for j in range(3)]
    vparts = []
    for k in range(4):
        d = x[:, 3 * k:3 * k + 3] - xp[:, 3 * k:3 * k + 3]
        proj = (d[:, 0:1] * rrow[0] + d[:, 1:2] * rrow[1]
                + d[:, 2:3] * rrow[2])
        direct, rbf = _decouple(proj)
        vparts += [direct, rbf]
    v = jnp.concatenate(vparts, axis=-1)
    v_ref[...] = jnp.where(jnp.isnan(v), jnp.zeros_like(v), v)
    f_ref[...] = jnp.concatenate(
        [r9, ca, x, jnp.zeros_like(x[:, 0:8])], axis=-1)


def _edge_body(gs_ref, gd_ref, sf_ref, df_ref, e_ref, rts_ref, tts_ref):
    gs = gs_ref[...]
    gd = gd_ref[...]
    ts = gs[:, 9:12]
    rts_rows = [
        sum(gd[:, 3 * j + i:3 * j + i + 1] * gs[:, 3 * j:3 * j + 3]
            for j in range(3))
        for i in range(3)
    ]
    rts_ref[...] = jnp.concatenate(rts_rows, axis=-1)
    dt = ts - gd[:, 9:12]
    tts = (dt[:, 0:1] * gd[:, 0:3] + dt[:, 1:2] * gd[:, 3:6]
           + dt[:, 2:3] * gd[:, 6:9])
    tts_ref[...] = tts
    eparts = []
    for k in range(8):
        g = gs if k < 4 else gd
        p = g[:, 12 + 3 * (k % 4):12 + 3 * (k % 4) + 3]
        q = p - ts
        proj = (q[:, 0:1] * gs[:, 0:3] + q[:, 1:2] * gs[:, 3:6]
                + q[:, 2:3] * gs[:, 6:9])
        direct, rbf = _decouple(proj)
        eparts += [direct, rbf]
    eparts.append(jnp.concatenate(
        [rts_rows[k][:, i:i + 1] for i in range(3) for k in range(3)],
        axis=-1))
    directT, rbfT = _decouple(tts)
    eparts += [directT, rbfT]
    d = sf_ref[...] - df_ref[...]
    ang = d * _freq_row()
    eparts += [jnp.cos(ang), jnp.sin(ang)]
    e_ref[...] = jnp.concatenate(eparts, axis=-1)


_NW = 32
_EPW = _E // _NW
_CH = 1000
_NCHUNK = _EPW // _CH


@functools.cache
def _build_gather_sc():
    info = plsc.get_sparse_core_info()
    nc = info.num_cores
    assert nc * info.num_subcores == _NW

    @functools.partial(
        pl.kernel,
        mesh=plsc.VectorSubcoreMesh(core_axis_name="c", subcore_axis_name="s"),
        out_type=[jax.ShapeDtypeStruct((_E, _DTAB), jnp.float32),
                  jax.ShapeDtypeStruct((_E, _DTAB), jnp.float32)],
        scratch_types=[pltpu.VMEM((_CH,), jnp.int32),
                       pltpu.VMEM((_CH, _DTAB), jnp.float32),
                       pltpu.VMEM((_CH,), jnp.int32),
                       pltpu.VMEM((_CH, _DTAB), jnp.float32),
                       pltpu.SemaphoreType.DMA,
                       pltpu.SemaphoreType.DMA],
        compiler_params=pltpu.CompilerParams(use_tc_tiling_on_sc=False),
    )
    def _gather_sc(table_hbm, src_hbm, dst_hbm, gsrc_hbm, gdst_hbm,
                   idxs_v, rows_s, idxd_v, rows_d, sem_s, sem_d):
        wid = lax.axis_index("s") * nc + lax.axis_index("c")

        def body(i, carry):
            base = wid * _EPW + i * _CH
            pltpu.sync_copy(src_hbm.at[pl.ds(base, _CH)], idxs_v)
            pltpu.sync_copy(dst_hbm.at[pl.ds(base, _CH)], idxd_v)
            cs = pltpu.async_copy(table_hbm.at[idxs_v], rows_s, sem_s)
            cd = pltpu.async_copy(table_hbm.at[idxd_v], rows_d, sem_d)
            cs.wait()
            cd.wait()
            pltpu.sync_copy(rows_s, gsrc_hbm.at[pl.ds(base, _CH)])
            pltpu.sync_copy(rows_d, gdst_hbm.at[pl.ds(base, _CH)])
            return carry

        lax.fori_loop(0, _NCHUNK, body, 0)

    return _gather_sc


def _gather_rows(table, src, dst):
    return _build_gather_sc()(table, src, dst)


def _node_call(xf, xprev):
    return pl.pallas_call(
        _node_body,
        grid=(_N // _NB,),
        in_specs=[pl.BlockSpec((_NB, 12), lambda i: (i, 0)),
                  pl.BlockSpec((_NB, 12), lambda i: (i, 0))],
        out_specs=[pl.BlockSpec((_NB, 9), lambda i: (i, 0)),
                   pl.BlockSpec((_NB, 3), lambda i: (i, 0)),
                   pl.BlockSpec((_NB, 76), lambda i: (i, 0)),
                   pl.BlockSpec((_NB, _DTAB), lambda i: (i, 0))],
        out_shape=[jax.ShapeDtypeStruct((_N, 9), jnp.float32),
                   jax.ShapeDtypeStruct((_N, 3), jnp.float32),
                   jax.ShapeDtypeStruct((_N, 76), jnp.float32),
                   jax.ShapeDtypeStruct((_N, _DTAB), jnp.float32)],
    )(xf, xprev)


def _edge_call(gsrc, gdst, sf, df):
    return pl.pallas_call(
        _edge_body,
        grid=(_E // _EB,),
        in_specs=[pl.BlockSpec((_EB, _DTAB), lambda i: (i, 0)),
                  pl.BlockSpec((_EB, _DTAB), lambda i: (i, 0)),
                  pl.BlockSpec((_EB, 1), lambda i: (i, 0)),
                  pl.BlockSpec((_EB, 1), lambda i: (i, 0))],
        out_specs=[pl.BlockSpec((_EB, 196), lambda i: (i, 0)),
                   pl.BlockSpec((_EB, 9), lambda i: (i, 0)),
                   pl.BlockSpec((_EB, 3), lambda i: (i, 0))],
        out_shape=[jax.ShapeDtypeStruct((_E, 196), jnp.float32),
                   jax.ShapeDtypeStruct((_E, 9), jnp.float32),
                   jax.ShapeDtypeStruct((_E, 3), jnp.float32)],
    )(gsrc, gdst, sf, df)


def kernel(X, edge_idx, batch_id, chain_encoding, virtual_frame_num):
    xf = X.reshape(_N, 12)
    flat = X.reshape(-1, 3)
    xprev = jnp.concatenate([flat[0:1], flat[:-1]], axis=0).reshape(_N, 12)
    r9, t, v, table = _node_call(xf, xprev)
    src, dst = edge_idx[0], edge_idx[1]
    gsrc, gdst = _gather_rows(table, src, dst)
    sf = src.astype(jnp.float32).reshape(_E, 1)
    df = dst.astype(jnp.float32).reshape(_E, 1)
    e, rts9, tts = _edge_call(gsrc, gdst, sf, df)
    return (v, e, r9.reshape(_N, 3, 3), t, rts9.reshape(_E, 3, 3), tts,
            batch_id, edge_idx, chain_encoding)


# speedup vs baseline: 7.2209x; 3.7141x over previous
import functools

import jax
import jax.numpy as jnp
import numpy as np
from jax import lax
from jax.experimental import pallas as pl
from jax.experimental.pallas import tpu as pltpu
from jax.experimental.pallas import tpu_sc as plsc

_N = 50000
_E = 800000
_NUM_RBF = 16
_NB = 2000
_EB = 3200
_DTAB = 32

_MU_STEP = np.float32(20.0 / (_NUM_RBF - 1))
_INV_SIG = np.float32(_NUM_RBF / 20.0)
_FREQ_COEF = np.float32(-2.0 * np.log(10000.0) / _NUM_RBF)


def _mu_row():
    i = lax.broadcasted_iota(jnp.int32, (1, _NUM_RBF), 1).astype(jnp.float32)
    return i * _MU_STEP


def _freq_row():
    i = lax.broadcasted_iota(jnp.int32, (1, 8), 1).astype(jnp.float32)
    return jnp.exp(i * _FREQ_COEF)


def _decouple(proj):
    nsq = jnp.sum(proj * proj, axis=-1, keepdims=True)
    norm = jnp.sqrt(nsq)
    direct = proj / (norm + 1e-6)
    z = (norm - _mu_row()) * _INV_SIG
    rbf = jnp.exp(-(z * z))
    return direct, rbf


def _node_body(x_ref, xprev_ref, r_ref, t_ref, v_ref, f_ref):
    x = x_ref[...]
    xp = xprev_ref[...]
    ca = x[:, 3:6]
    n = x[:, 0:3] - ca
    c = x[:, 6:9] - ca
    cx, cy, cz = c[:, 0:1], c[:, 1:2], c[:, 2:3]
    eps = 1e-20
    norm1 = jnp.sqrt(eps + cx * cx + cy * cy)
    sin1 = -cy / norm1
    cos1 = cx / norm1
    norm2 = jnp.sqrt(eps + cx * cx + cy * cy + cz * cz)
    sin2 = cz / norm2
    cos2 = jnp.sqrt(cx * cx + cy * cy) / norm2
    c1 = [[cos1, -sin1, 0.0], [sin1, cos1, 0.0], [0.0, 0.0, 1.0]]
    c2 = [[cos2, 0.0, sin2], [0.0, 1.0, 0.0], [-sin2, 0.0, cos2]]
    cr = [[sum(c2[i][j] * c1[j][k] for j in range(3)) for k in range(3)]
          for i in range(3)]
    nvec = [n[:, 0:1], n[:, 1:2], n[:, 2:3]]
    nrot = [sum(cr[i][j] * nvec[j] for j in range(3)) for i in range(3)]
    norm3 = jnp.sqrt(eps + nrot[1] * nrot[1] + nrot[2] * nrot[2])
    sinn = -nrot[2] / norm3
    cosn = nrot[1] / norm3
    nm = [[1.0, 0.0, 0.0], [0.0, cosn, -sinn], [0.0, sinn, cosn]]
    rots = [[sum(nm[i][j] * cr[j][k] for j in range(3)) for k in range(3)]
            for i in range(3)]
    ones = jnp.ones_like(cx)
    rcols = [rots[b][a] * ones for a in range(3) for b in range(3)]
    r9 = jnp.concatenate(rcols, axis=-1)
    r_ref[...] = r9
    t_ref[...] = ca
    rrow = [jnp.concatenate([rots[0][j] * ones, rots[1][j] * ones,
                             rots[2][j] * ones], axis=-1) for j in range(3)]
    vparts = []
    for k in range(4):
        d = x[:, 3 * k:3 * k + 3] - xp[:, 3 * k:3 * k + 3]
        proj = (d[:, 0:1] * rrow[0] + d[:, 1:2] * rrow[1]
                + d[:, 2:3] * rrow[2])
        direct, rbf = _decouple(proj)
        vparts += [direct, rbf]
    v = jnp.concatenate(vparts, axis=-1)
    v_ref[...] = jnp.where(jnp.isnan(v), jnp.zeros_like(v), v)
    xx = jnp.concatenate([x[:, 0:1], x[:, 3:4], x[:, 6:7], x[:, 9:10]], -1)
    xy = jnp.concatenate([x[:, 1:2], x[:, 4:5], x[:, 7:8], x[:, 10:11]], -1)
    xz = jnp.concatenate([x[:, 2:3], x[:, 5:6], x[:, 8:9], x[:, 11:12]], -1)
    f_ref[...] = jnp.concatenate(
        [r9, ca, xx, xy, xz, jnp.zeros_like(x[:, 0:8])], axis=-1)


def _mu_col():
    i = lax.broadcasted_iota(jnp.int32, (_NUM_RBF, 1), 0).astype(jnp.float32)
    return i * _MU_STEP


def _freq_col():
    i = lax.broadcasted_iota(jnp.int32, (8, 1), 0).astype(jnp.float32)
    return jnp.exp(i * _FREQ_COEF)


def _rbf_rows(norm1):
    z = (jnp.broadcast_to(norm1, (_NUM_RBF, norm1.shape[1])) - _mu_col()) \
        * _INV_SIG
    return jnp.exp(-(z * z))


def _edge_body(gs_ref, gd_ref, sf_ref, df_ref, e_ref, rts_ref, tts_ref):
    gst = gs_ref[...].T
    gdt = gd_ref[...].T
    rts_rows = [
        sum(gdt[3 * j + i:3 * j + i + 1] * gst[3 * j:3 * j + 3]
            for j in range(3))
        for i in range(3)
    ]
    rts_t = jnp.concatenate(
        rts_rows + [jnp.zeros_like(gst[0:7])], axis=0)
    rts_ref[...] = rts_t.T[:, :9]
    dt = gst[9:12] - gdt[9:12]
    tts = sum(dt[j:j + 1] * gdt[3 * j:3 * j + 3] for j in range(3))
    tts_pad = jnp.concatenate([tts, jnp.zeros_like(gst[0:5])], axis=0)
    tts_ref[...] = tts_pad.T[:, :3]
    qx = jnp.concatenate([gst[12:16], gdt[12:16]], axis=0) - gst[9:10]
    qy = jnp.concatenate([gst[16:20], gdt[16:20]], axis=0) - gst[10:11]
    qz = jnp.concatenate([gst[20:24], gdt[20:24]], axis=0) - gst[11:12]
    pj = [qx * gst[0 + i:1 + i] + qy * gst[3 + i:4 + i]
          + qz * gst[6 + i:7 + i] for i in range(3)]
    nsq = pj[0] * pj[0] + pj[1] * pj[1] + pj[2] * pj[2]
    norm = jnp.sqrt(nsq)
    inv = 1.0 / (norm + 1e-6)
    dirs = [p * inv for p in pj]
    erows = []
    for k in range(8):
        erows += [dirs[0][k:k + 1], dirs[1][k:k + 1], dirs[2][k:k + 1],
                  _rbf_rows(norm[k:k + 1])]
    erows.append(jnp.concatenate(
        [rts_rows[k][i:i + 1] for i in range(3) for k in range(3)], axis=0))
    tnsq = tts[0:1] * tts[0:1] + tts[1:2] * tts[1:2] + tts[2:3] * tts[2:3]
    tnorm = jnp.sqrt(tnsq)
    tinv = 1.0 / (tnorm + 1e-6)
    erows += [tts[0:1] * tinv, tts[1:2] * tinv, tts[2:3] * tinv,
              _rbf_rows(tnorm)]
    d = (sf_ref[...] - df_ref[...]).T
    ang = jnp.broadcast_to(d, (8, d.shape[1])) * _freq_col()
    erows += [jnp.cos(ang), jnp.sin(ang)]
    e_t = jnp.concatenate(
        erows + [jnp.zeros_like(qx[0:4])], axis=0)
    e_ref[...] = e_t.T[:, :196]


_NW = 32
_EPW = _E // _NW
_CH = 1000
_NCHUNK = _EPW // _CH


@functools.cache
def _build_gather_sc():
    info = plsc.get_sparse_core_info()
    nc = info.num_cores
    assert nc * info.num_subcores == _NW

    @functools.partial(
        pl.kernel,
        mesh=plsc.VectorSubcoreMesh(core_axis_name="c", subcore_axis_name="s"),
        out_type=[jax.ShapeDtypeStruct((_E, _DTAB), jnp.float32),
                  jax.ShapeDtypeStruct((_E, _DTAB), jnp.float32)],
        scratch_types=[pltpu.VMEM((_CH,), jnp.int32),
                       pltpu.VMEM((_CH, _DTAB), jnp.float32),
                       pltpu.VMEM((_CH,), jnp.int32),
                       pltpu.VMEM((_CH, _DTAB), jnp.float32),
                       pltpu.SemaphoreType.DMA,
                       pltpu.SemaphoreType.DMA],
        compiler_params=pltpu.CompilerParams(use_tc_tiling_on_sc=False),
    )
    def _gather_sc(table_hbm, src_hbm, dst_hbm, gsrc_hbm, gdst_hbm,
                   idxs_v, rows_s, idxd_v, rows_d, sem_s, sem_d):
        wid = lax.axis_index("s") * nc + lax.axis_index("c")

        def body(i, carry):
            base = wid * _EPW + i * _CH
            pltpu.sync_copy(src_hbm.at[pl.ds(base, _CH)], idxs_v)
            pltpu.sync_copy(dst_hbm.at[pl.ds(base, _CH)], idxd_v)
            cs = pltpu.async_copy(table_hbm.at[idxs_v], rows_s, sem_s)
            cd = pltpu.async_copy(table_hbm.at[idxd_v], rows_d, sem_d)
            cs.wait()
            cd.wait()
            pltpu.sync_copy(rows_s, gsrc_hbm.at[pl.ds(base, _CH)])
            pltpu.sync_copy(rows_d, gdst_hbm.at[pl.ds(base, _CH)])
            return carry

        lax.fori_loop(0, _NCHUNK, body, 0)

    return _gather_sc


def _gather_rows(table, src, dst):
    return _build_gather_sc()(table, src, dst)


def _node_call(xf, xprev):
    return pl.pallas_call(
        _node_body,
        grid=(_N // _NB,),
        in_specs=[pl.BlockSpec((_NB, 12), lambda i: (i, 0)),
                  pl.BlockSpec((_NB, 12), lambda i: (i, 0))],
        out_specs=[pl.BlockSpec((_NB, 9), lambda i: (i, 0)),
                   pl.BlockSpec((_NB, 3), lambda i: (i, 0)),
                   pl.BlockSpec((_NB, 76), lambda i: (i, 0)),
                   pl.BlockSpec((_NB, _DTAB), lambda i: (i, 0))],
        out_shape=[jax.ShapeDtypeStruct((_N, 9), jnp.float32),
                   jax.ShapeDtypeStruct((_N, 3), jnp.float32),
                   jax.ShapeDtypeStruct((_N, 76), jnp.float32),
                   jax.ShapeDtypeStruct((_N, _DTAB), jnp.float32)],
    )(xf, xprev)


def _edge_call(gsrc, gdst, sf, df):
    return pl.pallas_call(
        _edge_body,
        grid=(_E // _EB,),
        in_specs=[pl.BlockSpec((_EB, _DTAB), lambda i: (i, 0)),
                  pl.BlockSpec((_EB, _DTAB), lambda i: (i, 0)),
                  pl.BlockSpec((_EB, 1), lambda i: (i, 0)),
                  pl.BlockSpec((_EB, 1), lambda i: (i, 0))],
        out_specs=[pl.BlockSpec((_EB, 196), lambda i: (i, 0)),
                   pl.BlockSpec((_EB, 9), lambda i: (i, 0)),
                   pl.BlockSpec((_EB, 3), lambda i: (i, 0))],
        out_shape=[jax.ShapeDtypeStruct((_E, 196), jnp.float32),
                   jax.ShapeDtypeStruct((_E, 9), jnp.float32),
                   jax.ShapeDtypeStruct((_E, 3), jnp.float32)],
    )(gsrc, gdst, sf, df)


def kernel(X, edge_idx, batch_id, chain_encoding, virtual_frame_num):
    xf = X.reshape(_N, 12)
    flat = X.reshape(-1, 3)
    xprev = jnp.concatenate([flat[0:1], flat[:-1]], axis=0).reshape(_N, 12)
    r9, t, v, table = _node_call(xf, xprev)
    src, dst = edge_idx[0], edge_idx[1]
    gsrc, gdst = _gather_rows(table, src, dst)
    sf = src.astype(jnp.float32).reshape(_E, 1)
    df = dst.astype(jnp.float32).reshape(_E, 1)
    e, rts9, tts = _edge_call(gsrc, gdst, sf, df)
    return (v, e, r9.reshape(_N, 3, 3), t, rts9.reshape(_E, 3, 3), tts,
            batch_id, edge_idx, chain_encoding)
